# baseline (speedup 1.0000x reference)
import functools

import jax
import jax.numpy as jnp
from jax import lax
from jax.experimental import pallas as pl
from jax.experimental.pallas import tpu as pltpu
from jax.experimental.pallas import tpu_sc as plsc

_B = 1024
_M = 16
_C = 128
_NNZ = 512
_NT = 34

_LANES = 16
_NW = 32
_BPW = _B // _NW
_CCHUNKS = _C // _LANES
_MPTR_PAD = 32


def _sc_tensor_product(x1f, x2f, wf, cg, p12, paw, mptr_pad):
    mesh = plsc.VectorSubcoreMesh(core_axis_name="c", subcore_axis_name="s")

    @functools.partial(
        pl.kernel,
        mesh=mesh,
        out_type=jax.ShapeDtypeStruct((_B, _M * _C), jnp.float32),
        compiler_params=pltpu.CompilerParams(needs_layout_passes=False),
        scratch_types=[
            pltpu.SMEM((_NNZ,), jnp.int32),
            pltpu.SMEM((_NNZ,), jnp.int32),
            pltpu.SMEM((_NNZ,), jnp.float32),
            pltpu.SMEM((_MPTR_PAD,), jnp.int32),
            pltpu.VMEM((_NNZ,), jnp.int32),
            pltpu.VMEM((_NNZ,), jnp.int32),
            pltpu.VMEM((_NNZ,), jnp.float32),
            pltpu.VMEM((_MPTR_PAD,), jnp.int32),
            pltpu.VMEM((_M * _C,), jnp.float32),
            pltpu.VMEM((_M * _C,), jnp.float32),
            pltpu.VMEM((_NT * _C,), jnp.float32),
            pltpu.VMEM((_M * _C,), jnp.float32),
        ],
    )
    def k(x1_hbm, x2_hbm, w_hbm, cg_hbm, p12_hbm, paw_hbm, mptr_hbm,
          out_hbm, p12_s, paw_s, cg_s, mptr_s, p12_b, paw_b, cg_b, mptr_b,
          x1_v, x2_v, w_v, out_v):
        wid = lax.axis_index("c") * 16 + lax.axis_index("s")

        pltpu.sync_copy(p12_hbm, p12_b)
        pltpu.sync_copy(paw_hbm, paw_b)
        pltpu.sync_copy(cg_hbm, cg_b)
        pltpu.sync_copy(mptr_hbm, mptr_b)

        @plsc.parallel_loop(0, _NNZ, _LANES)
        def fill_body(base):
            v12 = p12_b[pl.ds(base, _LANES)]
            vaw = paw_b[pl.ds(base, _LANES)]
            vcg = cg_b[pl.ds(base, _LANES)]
            for j in range(_LANES):
                p12_s[base + j] = v12[j]
                paw_s[base + j] = vaw[j]
                cg_s[base + j] = vcg[j]

        @plsc.parallel_loop(0, _MPTR_PAD, _LANES)
        def fill_mptr(base):
            vmp = mptr_b[pl.ds(base, _LANES)]
            for j in range(_LANES):
                mptr_s[base + j] = vmp[j]

        def batch_body(i, carry):
            b = wid * _BPW + i
            pltpu.sync_copy(x1_hbm.at[b], x1_v)
            pltpu.sync_copy(x2_hbm.at[b], x2_v)
            pltpu.sync_copy(w_hbm.at[b], w_v)

            def seg_body(m, carry2):
                st = mptr_s[m]
                en = mptr_s[m + 1]
                zero = jnp.zeros((_LANES,), jnp.float32)
                init = (zero,) * _CCHUNKS

                @plsc.parallel_loop(st, en, 1, unroll=3, carry=init)
                def e_body(n, acc):
                    s12 = p12_s[n]
                    aws = paw_s[n]
                    cgs = cg_s[n]
                    o1 = s12 & 2047
                    o2 = lax.shift_right_logical(s12, 11)
                    new = []
                    for kk in range(_CCHUNKS):
                        g1 = x1_v[pl.ds(o1 + kk * _LANES, _LANES)]
                        g2 = x2_v[pl.ds(o2 + kk * _LANES, _LANES)]
                        gw = w_v[pl.ds(aws + kk * _LANES, _LANES)]
                        new.append(acc[kk] + g1 * g2 * gw * cgs)
                    return tuple(new)

                for kk in range(_CCHUNKS):
                    out_v[pl.ds(m * _C + kk * _LANES, _LANES)] = e_body[kk]
                return carry2
            lax.fori_loop(0, _M, seg_body, 0)

            pltpu.sync_copy(out_v, out_hbm.at[b])
            return carry
        lax.fori_loop(0, _BPW, batch_body, 0)

    return k(x1f, x2f, wf, cg, p12, paw, mptr_pad)


def kernel(x1, x2, weight, CG_vals, l_ind_M1M2, M1, M2, M_ptr_M1M2):
    a1 = M1 * _C
    a2 = M2 * _C
    aw = l_ind_M1M2 * _C
    p12 = a1 | (a2 << 11)
    mptr_pad = jnp.concatenate(
        [M_ptr_M1M2, jnp.zeros((_MPTR_PAD - _M - 1,), jnp.int32)])

    x1f = x1.reshape(_B, _M * _C)
    x2f = x2.reshape(_B, _M * _C)
    wf = weight.reshape(_B, _NT * _C)

    out = _sc_tensor_product(x1f, x2f, wf, CG_vals, p12, aw, mptr_pad)
    return out.reshape(_B, _M, _C)


# speedup vs baseline: 1.1030x; 1.1030x over previous
import functools

import jax
import jax.numpy as jnp
from jax import lax
from jax.experimental import pallas as pl
from jax.experimental.pallas import tpu as pltpu
from jax.experimental.pallas import tpu_sc as plsc

_B = 1024
_M = 16
_C = 128
_NNZ = 512
_NT = 34

_LANES = 16
_NW = 32
_BPW = _B // _NW
_CCHUNKS = _C // _LANES
_MPTR_PAD = 32


def _sc_tensor_product(x1f, x2f, wf, cg, p12, paw, mptr_pad):
    mesh = plsc.VectorSubcoreMesh(core_axis_name="c", subcore_axis_name="s")

    @functools.partial(
        pl.kernel,
        mesh=mesh,
        out_type=jax.ShapeDtypeStruct((_B, _M * _C), jnp.float32),
        compiler_params=pltpu.CompilerParams(needs_layout_passes=False),
        scratch_types=[
            pltpu.SMEM((_NNZ,), jnp.int32),
            pltpu.SMEM((_NNZ,), jnp.int32),
            pltpu.SMEM((_NNZ,), jnp.float32),
            pltpu.SMEM((_MPTR_PAD,), jnp.int32),
            pltpu.VMEM((_NNZ,), jnp.int32),
            pltpu.VMEM((_NNZ,), jnp.int32),
            pltpu.VMEM((_NNZ,), jnp.float32),
            pltpu.VMEM((_MPTR_PAD,), jnp.int32),
            pltpu.VMEM((_M * _C,), jnp.float32),
            pltpu.VMEM((_M * _C,), jnp.float32),
            pltpu.VMEM((_NT * _C,), jnp.float32),
            pltpu.VMEM((_M * _C,), jnp.float32),
        ],
    )
    def k(x1_hbm, x2_hbm, w_hbm, cg_hbm, p12_hbm, paw_hbm, mptr_hbm,
          out_hbm, p12_s, paw_s, cg_s, mptr_s, p12_b, paw_b, cg_b, mptr_b,
          x1_v, x2_v, w_v, out_v):
        wid = lax.axis_index("c") * 16 + lax.axis_index("s")

        pltpu.sync_copy(p12_hbm, p12_b)
        pltpu.sync_copy(paw_hbm, paw_b)
        pltpu.sync_copy(cg_hbm, cg_b)
        pltpu.sync_copy(mptr_hbm, mptr_b)

        @plsc.parallel_loop(0, _NNZ, _LANES)
        def fill_body(base):
            v12 = p12_b[pl.ds(base, _LANES)]
            vaw = paw_b[pl.ds(base, _LANES)]
            vcg = cg_b[pl.ds(base, _LANES)]
            for j in range(_LANES):
                p12_s[base + j] = v12[j]
                paw_s[base + j] = vaw[j]
                cg_s[base + j] = vcg[j]

        @plsc.parallel_loop(0, _MPTR_PAD, _LANES)
        def fill_mptr(base):
            vmp = mptr_b[pl.ds(base, _LANES)]
            for j in range(_LANES):
                mptr_s[base + j] = vmp[j]

        def batch_body(i, carry):
            b = wid * _BPW + i
            pltpu.sync_copy(x1_hbm.at[b], x1_v)
            pltpu.sync_copy(x2_hbm.at[b], x2_v)
            pltpu.sync_copy(w_hbm.at[b], w_v)

            def seg_body(m, carry2):
                st = mptr_s[m]
                en = mptr_s[m + 1]
                zero = jnp.zeros((_LANES,), jnp.float32)
                init = (zero,) * _CCHUNKS

                @plsc.parallel_loop(st, en, 1, unroll=2, carry=init)
                def e_body(n, acc):
                    s12 = p12_s[n]
                    aws = paw_s[n]
                    cgs = cg_s[n]
                    o1 = s12 & 2047
                    o2 = lax.shift_right_logical(s12, 11)
                    new = []
                    for kk in range(_CCHUNKS):
                        g1 = x1_v[pl.ds(o1 + kk * _LANES, _LANES)]
                        g2 = x2_v[pl.ds(o2 + kk * _LANES, _LANES)]
                        gw = w_v[pl.ds(aws + kk * _LANES, _LANES)]
                        new.append(acc[kk] + g1 * g2 * gw * cgs)
                    return tuple(new)

                for kk in range(_CCHUNKS):
                    out_v[pl.ds(m * _C + kk * _LANES, _LANES)] = e_body[kk]
                return carry2
            lax.fori_loop(0, _M, seg_body, 0)

            pltpu.sync_copy(out_v, out_hbm.at[b])
            return carry
        lax.fori_loop(0, _BPW, batch_body, 0)

    return k(x1f, x2f, wf, cg, p12, paw, mptr_pad)


def kernel(x1, x2, weight, CG_vals, l_ind_M1M2, M1, M2, M_ptr_M1M2):
    a1 = M1 * _C
    a2 = M2 * _C
    aw = l_ind_M1M2 * _C
    p12 = a1 | (a2 << 11)
    mptr_pad = jnp.concatenate(
        [M_ptr_M1M2, jnp.zeros((_MPTR_PAD - _M - 1,), jnp.int32)])

    x1f = x1.reshape(_B, _M * _C)
    x2f = x2.reshape(_B, _M * _C)
    wf = weight.reshape(_B, _NT * _C)

    out = _sc_tensor_product(x1f, x2f, wf, CG_vals, p12, aw, mptr_pad)
    return out.reshape(_B, _M, _C)


# speedup vs baseline: 1.2044x; 1.0920x over previous
import functools

import jax
import jax.numpy as jnp
from jax import lax
from jax.experimental import pallas as pl
from jax.experimental.pallas import tpu as pltpu
from jax.experimental.pallas import tpu_sc as plsc

_B = 1024
_M = 16
_C = 128
_NNZ = 512
_NT = 34

_LANES = 16
_NW = 32
_BPW = _B // _NW
_CCHUNKS = _C // _LANES
_MPTR_PAD = 32


def _sc_tensor_product(x1, x2, w, cg, p12, paw, mptr_pad):
    mesh = plsc.VectorSubcoreMesh(core_axis_name="c", subcore_axis_name="s")

    @functools.partial(
        pl.kernel,
        mesh=mesh,
        out_type=jax.ShapeDtypeStruct((_B, _M, _C), jnp.float32),
        compiler_params=pltpu.CompilerParams(needs_layout_passes=False),
        scratch_types=[
            pltpu.SMEM((_NNZ,), jnp.int32),
            pltpu.SMEM((_NNZ,), jnp.int32),
            pltpu.SMEM((_NNZ,), jnp.float32),
            pltpu.SMEM((_MPTR_PAD,), jnp.int32),
            pltpu.VMEM((_NNZ,), jnp.int32),
            pltpu.VMEM((_NNZ,), jnp.int32),
            pltpu.VMEM((_NNZ,), jnp.float32),
            pltpu.VMEM((_MPTR_PAD,), jnp.int32),
            pltpu.VMEM((_M, _C), jnp.float32),
            pltpu.VMEM((_M, _C), jnp.float32),
            pltpu.VMEM((_NT, _C), jnp.float32),
            pltpu.VMEM((_M, _C), jnp.float32),
        ],
    )
    def k(x1_hbm, x2_hbm, w_hbm, cg_hbm, p12_hbm, paw_hbm, mptr_hbm,
          out_hbm, p12_s, paw_s, cg_s, mptr_s, p12_b, paw_b, cg_b, mptr_b,
          x1_v, x2_v, w_v, out_v):
        wid = lax.axis_index("c") * 16 + lax.axis_index("s")

        pltpu.sync_copy(p12_hbm, p12_b)
        pltpu.sync_copy(paw_hbm, paw_b)
        pltpu.sync_copy(cg_hbm, cg_b)
        pltpu.sync_copy(mptr_hbm, mptr_b)

        @plsc.parallel_loop(0, _NNZ, _LANES)
        def fill_body(base):
            v12 = p12_b[pl.ds(base, _LANES)]
            vaw = paw_b[pl.ds(base, _LANES)]
            vcg = cg_b[pl.ds(base, _LANES)]
            for j in range(_LANES):
                p12_s[base + j] = v12[j]
                paw_s[base + j] = vaw[j]
                cg_s[base + j] = vcg[j]

        @plsc.parallel_loop(0, _MPTR_PAD, _LANES)
        def fill_mptr(base):
            vmp = mptr_b[pl.ds(base, _LANES)]
            for j in range(_LANES):
                mptr_s[base + j] = vmp[j]

        def batch_body(i, carry):
            b = wid * _BPW + i
            pltpu.sync_copy(x1_hbm.at[b], x1_v)
            pltpu.sync_copy(x2_hbm.at[b], x2_v)
            pltpu.sync_copy(w_hbm.at[b], w_v)

            def seg_body(m, carry2):
                st = mptr_s[m]
                en = mptr_s[m + 1]
                zero = jnp.zeros((_LANES,), jnp.float32)
                init = (zero,) * _CCHUNKS

                @plsc.parallel_loop(st, en, 1, unroll=2, carry=init)
                def acc_fin(n, acc):
                    s12 = p12_s[n]
                    aws = paw_s[n]
                    cgs = cg_s[n]
                    o1 = s12 & 255
                    o2 = lax.shift_right_logical(s12, 8)
                    new = []
                    for kk in range(_CCHUNKS):
                        g1 = x1_v[o1, pl.ds(kk * _LANES, _LANES)]
                        g2 = x2_v[o2, pl.ds(kk * _LANES, _LANES)]
                        gw = w_v[aws, pl.ds(kk * _LANES, _LANES)]
                        new.append(acc[kk] + g1 * g2 * gw * cgs)
                    return tuple(new)

                for kk in range(_CCHUNKS):
                    out_v[m, pl.ds(kk * _LANES, _LANES)] = acc_fin[kk]
                return carry2
            lax.fori_loop(0, _M, seg_body, 0)

            pltpu.sync_copy(out_v, out_hbm.at[b])
            return carry
        lax.fori_loop(0, _BPW, batch_body, 0)

    return k(x1, x2, w, cg, p12, paw, mptr_pad)


def kernel(x1, x2, weight, CG_vals, l_ind_M1M2, M1, M2, M_ptr_M1M2):
    p12 = M1 | (M2 << 8)
    mptr_pad = jnp.concatenate(
        [M_ptr_M1M2, jnp.zeros((_MPTR_PAD - _M - 1,), jnp.int32)])
    return _sc_tensor_product(x1, x2, weight, CG_vals, p12, l_ind_M1M2,
                              mptr_pad)
